# baseline (speedup 1.0000x reference)
import jax
import jax.numpy as jnp
from jax.experimental import pallas as pl
from jax.experimental.pallas import tpu as pltpu

TILE_VR = 4096


def _matmul_nt_body(h_ref, wt_ref, o_ref):
    o_ref[...] = jax.lax.dot_general(
        h_ref[...], wt_ref[...],
        dimension_numbers=(((1,), (1,)), ((), ())),
        preferred_element_type=jnp.float32)


@jax.jit
def kernel(x, embed, w_out):
    b, s = x.shape
    n_tok = b * s
    vocab = w_out.shape[1]
    d = embed.shape[1]
    idx = x.reshape(n_tok)

    h = jnp.take(embed, idx, axis=0)
    w_t = w_out.T

    n_v = pl.cdiv(vocab, TILE_VR)
    logits = pl.pallas_call(
        _matmul_nt_body,
        grid=(n_v,),
        in_specs=[
            pl.BlockSpec((n_tok, d), lambda v: (0, 0)),
            pl.BlockSpec((TILE_VR, d), lambda v: (v, 0)),
        ],
        out_specs=pl.BlockSpec((n_tok, TILE_VR), lambda v: (0, v)),
        out_shape=jax.ShapeDtypeStruct((n_tok, vocab), jnp.float32),
        compiler_params=pltpu.CompilerParams(
            dimension_semantics=("arbitrary",),
        ),
    )(h, w_t)

    return logits.reshape(b, s, vocab)
